# baseline (speedup 1.0000x reference)
import functools

import jax
import jax.numpy as jnp
from jax import lax
from jax.experimental import pallas as pl
from jax.experimental.pallas import tpu as pltpu

_UNROLL = 8
_W3_CHUNK = 16


def _body(rows, nbuf, ids_ref, table_ref, W3_hbm, W1_ref, b1_ref, W2_ref,
          b2_ref, b3_ref, out_ref, e_ref, h2_ref, w3f_ref, w3b_ref, obuf_ref,
          gsem, wsem, osem):
    B = e_ref.shape[0]
    D = e_ref.shape[1]
    i = pl.program_id(0)
    nt = B // rows
    slot = lax.rem(i, nbuf)

    @pl.when(i == 0)
    def _():
        def issue(r, c):
            for j in range(_UNROLL):
                k = r * _UNROLL + j
                row = ids_ref[k]
                pltpu.make_async_copy(
                    table_ref.at[pl.ds(row, 1), :],
                    e_ref.at[pl.ds(k, 1), :],
                    gsem,
                ).start()
            return c

        lax.fori_loop(0, B // _UNROLL, issue, 0)

        n_chunks = W3_hbm.shape[0] // _W3_CHUNK

        def _w3_copy(c):
            return pltpu.make_async_copy(
                W3_hbm.at[pl.ds(c * _W3_CHUNK, _W3_CHUNK), :],
                w3f_ref.at[c % 2],
                wsem.at[c % 2],
            )

        _w3_copy(0).start()
        _w3_copy(1).start()
        for c in range(n_chunks):
            _w3_copy(c).wait()
            w3b_ref[pl.ds(c * _W3_CHUNK, _W3_CHUNK), :] = (
                w3f_ref[c % 2].astype(jnp.bfloat16))
            if c + 2 < n_chunks:
                _w3_copy(c + 2).start()

        pltpu.make_async_copy(table_ref.at[pl.ds(0, B), :], e_ref, gsem).wait()
        h1 = jnp.dot(e_ref[...], W1_ref[...],
                     preferred_element_type=jnp.float32) + b1_ref[...]
        h1 = jnp.maximum(h1, 0.0)
        h2 = jnp.dot(h1, W2_ref[...],
                     preferred_element_type=jnp.float32) + b2_ref[...]
        h2_ref[...] = jnp.maximum(h2, 0.0)

    def _copy(s, idx):
        base = pl.multiple_of(idx * rows, rows)
        return pltpu.make_async_copy(
            obuf_ref.at[s],
            out_ref.at[pl.ds(base, rows), :],
            osem.at[s],
        )

    @pl.when(i >= nbuf)
    def _():
        _copy(slot, i - nbuf).wait()

    r0 = pl.multiple_of(i * rows, rows)
    h2c = h2_ref[pl.ds(r0, rows), :].astype(jnp.bfloat16)
    obuf_ref[slot] = jnp.dot(h2c, w3b_ref[...],
                             preferred_element_type=jnp.float32) + b3_ref[...]
    for s in range(nbuf):
        @pl.when(slot == s)
        def _(s=s):
            _copy(s, i).start(priority=s % 2)

    @pl.when(i == nt - 1)
    def _():
        for idx in range(nt - nbuf, nt):
            _copy(idx % nbuf, idx).wait()


@functools.partial(jax.jit, static_argnames=("rows", "nbuf"))
def _fused(user_ids, table, W1, b1, W2, b2, W3, b3, rows=8, nbuf=4):
    B = user_ids.shape[0]
    D = table.shape[1]
    H1 = W1.shape[1]
    H2 = W2.shape[1]
    N = W3.shape[1]
    grid = (B // rows,)
    grid_spec = pltpu.PrefetchScalarGridSpec(
        num_scalar_prefetch=1,
        grid=grid,
        in_specs=[
            pl.BlockSpec(memory_space=pltpu.HBM),
            pl.BlockSpec(memory_space=pltpu.HBM),
            pl.BlockSpec((D, H1), lambda i, ids: (0, 0)),
            pl.BlockSpec((1, H1), lambda i, ids: (0, 0)),
            pl.BlockSpec((H1, H2), lambda i, ids: (0, 0)),
            pl.BlockSpec((1, H2), lambda i, ids: (0, 0)),
            pl.BlockSpec((1, N), lambda i, ids: (0, 0)),
        ],
        out_specs=pl.BlockSpec(memory_space=pltpu.HBM),
        scratch_shapes=[
            pltpu.VMEM((B, D), jnp.float32),
            pltpu.VMEM((B, H2), jnp.float32),
            pltpu.VMEM((2, _W3_CHUNK, N), jnp.float32),
            pltpu.VMEM((D, N), jnp.bfloat16),
            pltpu.VMEM((nbuf, rows, N), jnp.float32),
            pltpu.SemaphoreType.DMA,
            pltpu.SemaphoreType.DMA((2,)),
            pltpu.SemaphoreType.DMA((nbuf,)),
        ],
    )
    return pl.pallas_call(
        functools.partial(_body, rows, nbuf),
        grid_spec=grid_spec,
        out_shape=jax.ShapeDtypeStruct((B, N), jnp.float32),
        compiler_params=pltpu.CompilerParams(
            dimension_semantics=("arbitrary",),
        ),
    )(user_ids.astype(jnp.int32), table, W3, W1, b1.reshape(1, H1), W2,
      b2.reshape(1, H2), b3.reshape(1, N))


def kernel(user_ids, table, W1, b1, W2, b2, W3, b3):
    return _fused(user_ids, table, W1, b1, W2, b2, W3, b3)


# speedup vs baseline: 1.0734x; 1.0734x over previous
import functools

import jax
import jax.numpy as jnp
from jax import lax
from jax.experimental import pallas as pl
from jax.experimental.pallas import tpu as pltpu

_UNROLL = 8
_W3_CHUNK = 16


def _body(rows, nbuf, ids_ref, table_ref, W3_hbm, W1_ref, b1_ref, W2_ref,
          b2_ref, b3_ref, out_ref, e_ref, h2_ref, w3f_ref, w3b_ref, obuf_ref,
          gsem, wsem, osem):
    B = e_ref.shape[0]
    D = e_ref.shape[1]
    i = pl.program_id(0)
    nt = B // rows
    slot = lax.rem(i, nbuf)

    @pl.when(i == 0)
    def _():
        def issue(r, c):
            for j in range(_UNROLL):
                k = r * _UNROLL + j
                row = ids_ref[k]
                pltpu.make_async_copy(
                    table_ref.at[pl.ds(row, 1), :],
                    e_ref.at[pl.ds(k, 1), :],
                    gsem,
                ).start()
            return c

        lax.fori_loop(0, B // _UNROLL, issue, 0)

        n_chunks = W3_hbm.shape[0] // _W3_CHUNK

        def _w3_copy(c):
            return pltpu.make_async_copy(
                W3_hbm.at[pl.ds(c * _W3_CHUNK, _W3_CHUNK), :],
                w3f_ref.at[c % 2],
                wsem.at[c % 2],
            )

        _w3_copy(0).start()
        _w3_copy(1).start()
        for c in range(n_chunks):
            _w3_copy(c).wait()
            w3b_ref[pl.ds(c * _W3_CHUNK, _W3_CHUNK), :] = (
                w3f_ref[c % 2].astype(jnp.bfloat16))
            if c + 2 < n_chunks:
                _w3_copy(c + 2).start()

        pltpu.make_async_copy(table_ref.at[pl.ds(0, B), :], e_ref, gsem).wait()
        h1 = jnp.dot(e_ref[...], W1_ref[...],
                     preferred_element_type=jnp.float32) + b1_ref[...]
        h1 = jnp.maximum(h1, 0.0)
        h2 = jnp.dot(h1, W2_ref[...],
                     preferred_element_type=jnp.float32) + b2_ref[...]
        h2_ref[...] = jnp.maximum(h2, 0.0)

    def _copy(s, idx):
        base = pl.multiple_of(idx * rows, rows)
        return pltpu.make_async_copy(
            obuf_ref.at[s],
            out_ref.at[pl.ds(base, rows), :],
            osem.at[s],
        )

    @pl.when(i >= nbuf)
    def _():
        _copy(slot, i - nbuf).wait()

    r0 = pl.multiple_of(i * rows, rows)
    h2c = h2_ref[pl.ds(r0, rows), :].astype(jnp.bfloat16)
    obuf_ref[slot] = jnp.dot(h2c, w3b_ref[...],
                             preferred_element_type=jnp.float32) + b3_ref[...]
    for s in range(nbuf):
        @pl.when(slot == s)
        def _(s=s):
            _copy(s, i).start(priority=s % 2)

    @pl.when(i == nt - 1)
    def _():
        for idx in range(nt - nbuf, nt):
            _copy(idx % nbuf, idx).wait()


@functools.partial(jax.jit, static_argnames=("rows", "nbuf"))
def _fused(user_ids, table, W1, b1, W2, b2, W3, b3, rows=32, nbuf=2):
    B = user_ids.shape[0]
    D = table.shape[1]
    H1 = W1.shape[1]
    H2 = W2.shape[1]
    N = W3.shape[1]
    grid = (B // rows,)
    grid_spec = pltpu.PrefetchScalarGridSpec(
        num_scalar_prefetch=1,
        grid=grid,
        in_specs=[
            pl.BlockSpec(memory_space=pltpu.HBM),
            pl.BlockSpec(memory_space=pltpu.HBM),
            pl.BlockSpec((D, H1), lambda i, ids: (0, 0)),
            pl.BlockSpec((1, H1), lambda i, ids: (0, 0)),
            pl.BlockSpec((H1, H2), lambda i, ids: (0, 0)),
            pl.BlockSpec((1, H2), lambda i, ids: (0, 0)),
            pl.BlockSpec((1, N), lambda i, ids: (0, 0)),
        ],
        out_specs=pl.BlockSpec(memory_space=pltpu.HBM),
        scratch_shapes=[
            pltpu.VMEM((B, D), jnp.float32),
            pltpu.VMEM((B, H2), jnp.float32),
            pltpu.VMEM((2, _W3_CHUNK, N), jnp.float32),
            pltpu.VMEM((D, N), jnp.bfloat16),
            pltpu.VMEM((nbuf, rows, N), jnp.float32),
            pltpu.SemaphoreType.DMA,
            pltpu.SemaphoreType.DMA((2,)),
            pltpu.SemaphoreType.DMA((nbuf,)),
        ],
    )
    return pl.pallas_call(
        functools.partial(_body, rows, nbuf),
        grid_spec=grid_spec,
        out_shape=jax.ShapeDtypeStruct((B, N), jnp.float32),
        compiler_params=pltpu.CompilerParams(
            dimension_semantics=("arbitrary",),
        ),
    )(user_ids.astype(jnp.int32), table, W3, W1, b1.reshape(1, H1), W2,
      b2.reshape(1, H2), b3.reshape(1, N))


def kernel(user_ids, table, W1, b1, W2, b2, W3, b3):
    return _fused(user_ids, table, W1, b1, W2, b2, W3, b3)


# speedup vs baseline: 1.1332x; 1.0557x over previous
import functools

import jax
import jax.numpy as jnp
from jax import lax
from jax.experimental import pallas as pl
from jax.experimental.pallas import tpu as pltpu

_UNROLL = 8


def _body(ids_ref, table_ref, W1_ref, b1_ref, W2_ref, b2_ref, W3_ref, b3_ref,
          out_ref, e_ref, h2_ref, gsem):
    B = e_ref.shape[0]

    @pl.when(pl.program_id(0) == 0)
    def _():
        def issue(r, c):
            for j in range(_UNROLL):
                k = r * _UNROLL + j
                row = ids_ref[k]
                pltpu.make_async_copy(
                    table_ref.at[pl.ds(row, 1), :],
                    e_ref.at[pl.ds(k, 1), :],
                    gsem,
                ).start()
            return c

        lax.fori_loop(0, B // _UNROLL, issue, 0)
        pltpu.make_async_copy(table_ref.at[pl.ds(0, B), :], e_ref, gsem).wait()

        h1 = jnp.dot(e_ref[...], W1_ref[...],
                     preferred_element_type=jnp.float32) + b1_ref[...]
        h1 = jnp.maximum(h1, 0.0)
        h2 = jnp.dot(h1, W2_ref[...],
                     preferred_element_type=jnp.float32) + b2_ref[...]
        h2_ref[...] = jnp.maximum(h2, 0.0).astype(jnp.bfloat16)

    w3 = W3_ref[...].astype(jnp.bfloat16)
    out_ref[...] = jnp.dot(h2_ref[...], w3,
                           preferred_element_type=jnp.float32) + b3_ref[...]


@functools.partial(jax.jit, static_argnames=("tile_n",))
def _fused(user_ids, table, W1, b1, W2, b2, W3, b3, tile_n=6144):
    B = user_ids.shape[0]
    D = table.shape[1]
    H1 = W1.shape[1]
    H2 = W2.shape[1]
    N = W3.shape[1]
    grid = (pl.cdiv(N, tile_n),)
    grid_spec = pltpu.PrefetchScalarGridSpec(
        num_scalar_prefetch=1,
        grid=grid,
        in_specs=[
            pl.BlockSpec(memory_space=pltpu.HBM),
            pl.BlockSpec((D, H1), lambda i, ids: (0, 0)),
            pl.BlockSpec((1, H1), lambda i, ids: (0, 0)),
            pl.BlockSpec((H1, H2), lambda i, ids: (0, 0)),
            pl.BlockSpec((1, H2), lambda i, ids: (0, 0)),
            pl.BlockSpec((D, tile_n), lambda i, ids: (0, i)),
            pl.BlockSpec((1, tile_n), lambda i, ids: (0, i)),
        ],
        out_specs=pl.BlockSpec((B, tile_n), lambda i, ids: (0, i)),
        scratch_shapes=[
            pltpu.VMEM((B, D), jnp.float32),
            pltpu.VMEM((B, H2), jnp.bfloat16),
            pltpu.SemaphoreType.DMA,
        ],
    )
    return pl.pallas_call(
        _body,
        grid_spec=grid_spec,
        out_shape=jax.ShapeDtypeStruct((B, N), jnp.float32),
        compiler_params=pltpu.CompilerParams(
            dimension_semantics=("arbitrary",),
        ),
    )(user_ids.astype(jnp.int32), table, W1, b1.reshape(1, H1), W2,
      b2.reshape(1, H2), W3, b3.reshape(1, N))


def kernel(user_ids, table, W1, b1, W2, b2, W3, b3):
    return _fused(user_ids, table, W1, b1, W2, b2, W3, b3)


# speedup vs baseline: 1.3572x; 1.1976x over previous
import functools

import jax
import jax.numpy as jnp
from jax import lax
from jax.experimental import pallas as pl
from jax.experimental.pallas import tpu as pltpu
from jax.sharding import Mesh, PartitionSpec as P

_UNROLL = 8
_W3_CHUNK = 16


def _gather_body(ids_ref, table_ref, mask_ref, e_ref, er_ref, gsem):
    B = er_ref.shape[0]

    def issue(r, c):
        for j in range(_UNROLL):
            k = r * _UNROLL + j
            row = ids_ref[k]
            pltpu.make_async_copy(
                table_ref.at[pl.ds(row, 1), :],
                er_ref.at[pl.ds(k, 1), :],
                gsem,
            ).start()
        return c

    lax.fori_loop(0, B // _UNROLL, issue, 0)
    pltpu.make_async_copy(table_ref.at[pl.ds(0, B), :], er_ref, gsem).wait()
    e_ref[...] = er_ref[...] * mask_ref[...]


def _gather(ids, table, mask2d):
    B = ids.shape[0]
    D = table.shape[1]
    grid_spec = pltpu.PrefetchScalarGridSpec(
        num_scalar_prefetch=1,
        grid=(1,),
        in_specs=[
            pl.BlockSpec(memory_space=pltpu.HBM),
            pl.BlockSpec((B, 1), lambda i, ids: (0, 0)),
        ],
        out_specs=pl.BlockSpec((B, D), lambda i, ids: (0, 0)),
        scratch_shapes=[
            pltpu.VMEM((B, D), jnp.float32),
            pltpu.SemaphoreType.DMA,
        ],
    )
    return pl.pallas_call(
        _gather_body,
        grid_spec=grid_spec,
        out_shape=jax.ShapeDtypeStruct((B, D), jnp.float32),
    )(ids, table, mask2d)


def _proj_body(rows, nbuf, nt, e_ref, W1_ref, b1_ref, W2_ref, b2_ref, W3_hbm,
               b3_ref, out_ref, h2_ref, w3f_ref, w3b_ref, obuf_ref, wsem,
               osem):
    i = pl.program_id(0)
    slot = lax.rem(i, nbuf)

    @pl.when(i == 0)
    def _():
        n_chunks = W3_hbm.shape[0] // _W3_CHUNK

        def _w3_copy(c):
            return pltpu.make_async_copy(
                W3_hbm.at[pl.ds(c * _W3_CHUNK, _W3_CHUNK), :],
                w3f_ref.at[c % 2],
                wsem.at[c % 2],
            )

        _w3_copy(0).start()
        _w3_copy(1).start()
        for c in range(n_chunks):
            _w3_copy(c).wait()
            w3b_ref[pl.ds(c * _W3_CHUNK, _W3_CHUNK), :] = (
                w3f_ref[c % 2].astype(jnp.bfloat16))
            if c + 2 < n_chunks:
                _w3_copy(c + 2).start()

        h1 = jnp.dot(e_ref[...], W1_ref[...],
                     preferred_element_type=jnp.float32) + b1_ref[...]
        h1 = jnp.maximum(h1, 0.0)
        h2 = jnp.dot(h1, W2_ref[...],
                     preferred_element_type=jnp.float32) + b2_ref[...]
        h2_ref[...] = jnp.maximum(h2, 0.0)

    def _copy(s, idx):
        base = pl.multiple_of(idx * rows, rows)
        return pltpu.make_async_copy(
            obuf_ref.at[s],
            out_ref.at[pl.ds(base, rows), :],
            osem.at[s],
        )

    @pl.when(i >= nbuf)
    def _():
        _copy(slot, i - nbuf).wait()

    r0 = pl.multiple_of(i * rows, rows)
    h2c = h2_ref[pl.ds(r0, rows), :].astype(jnp.bfloat16)
    obuf_ref[slot] = jnp.dot(h2c, w3b_ref[...],
                             preferred_element_type=jnp.float32) + b3_ref[...]

    for s in range(nbuf):
        @pl.when(slot == s)
        def _(s=s):
            _copy(s, i).start(priority=s % 2)

    @pl.when(i == nt - 1)
    def _():
        for idx in range(nt - nbuf, nt):
            _copy(idx % nbuf, idx).wait()


def _proj(e, W1, b1, W2, b2, W3s, b3s, rows=32, nbuf=4):
    B, D = e.shape
    H1 = W1.shape[1]
    H2 = W2.shape[1]
    N = W3s.shape[1]
    nt = B // rows
    grid_spec = pltpu.PrefetchScalarGridSpec(
        num_scalar_prefetch=0,
        grid=(nt,),
        in_specs=[
            pl.BlockSpec((B, D), lambda i: (0, 0)),
            pl.BlockSpec((D, H1), lambda i: (0, 0)),
            pl.BlockSpec((1, H1), lambda i: (0, 0)),
            pl.BlockSpec((H1, H2), lambda i: (0, 0)),
            pl.BlockSpec((1, H2), lambda i: (0, 0)),
            pl.BlockSpec(memory_space=pltpu.HBM),
            pl.BlockSpec((1, N), lambda i: (0, 0)),
        ],
        out_specs=pl.BlockSpec(memory_space=pltpu.HBM),
        scratch_shapes=[
            pltpu.VMEM((B, H2), jnp.float32),
            pltpu.VMEM((2, _W3_CHUNK, N), jnp.float32),
            pltpu.VMEM((H2, N), jnp.bfloat16),
            pltpu.VMEM((nbuf, rows, N), jnp.float32),
            pltpu.SemaphoreType.DMA((2,)),
            pltpu.SemaphoreType.DMA((nbuf,)),
        ],
    )
    return pl.pallas_call(
        functools.partial(_proj_body, rows, nbuf, nt),
        grid_spec=grid_spec,
        out_shape=jax.ShapeDtypeStruct((B, N), jnp.float32),
        compiler_params=pltpu.CompilerParams(
            dimension_semantics=("arbitrary",),
        ),
    )(e, W1, b1.reshape(1, H1), W2, b2.reshape(1, H2), W3s, b3s)


def _sharded_fn(user_ids, table, W1, b1, W2, b2, W3s, b3s):
    B = user_ids.shape[0]
    Vs = table.shape[0]
    ids32 = user_ids.astype(jnp.int32)
    lo = (lax.axis_index("x") * Vs).astype(jnp.int32)
    ids_local = ids32 - lo
    valid = (ids_local >= 0) & (ids_local < Vs)
    clamped = jnp.clip(ids_local, 0, Vs - 1)
    mask2d = valid.astype(jnp.float32).reshape(B, 1)
    e_loc = _gather(clamped, table, mask2d)
    e = lax.psum(e_loc, "x")
    return _proj(e, W1, b1, W2, b2, W3s, b3s)


def kernel(user_ids, table, W1, b1, W2, b2, W3, b3):
    N = W3.shape[1]
    b3r = b3.reshape(1, N)
    devs = jax.devices()
    if len(devs) >= 2 and N % 2 == 0 and table.shape[0] % 2 == 0:
        mesh = Mesh(devs[:2], ("x",))
        fn = jax.shard_map(
            _sharded_fn, mesh=mesh,
            in_specs=(P(), P("x", None), P(), P(), P(), P(),
                      P(None, "x"), P(None, "x")),
            out_specs=P(None, "x"),
            check_vma=False,
        )
        return fn(user_ids, table, W1, b1, W2, b2, W3, b3r)
    ids32 = user_ids.astype(jnp.int32)
    mask2d = jnp.ones((user_ids.shape[0], 1), jnp.float32)
    e = _gather(ids32, table, mask2d)
    return _proj(e, W1, b1, W2, b2, W3, b3r)
